# baseline (speedup 1.0000x reference)
import functools

import jax
import jax.numpy as jnp
from jax import lax
from jax.experimental import pallas as pl
from jax.experimental.pallas import tpu as pltpu
from jax.experimental.pallas import tpu_sc as plsc

_N = 10000
_E = 320000
_NHID = 16
_DEPTH = 3

_NW = 32
_KC = 100
_EPW = _E // _NW
_NCH = _EPW // _KC
_NP = 10240

_BN = 2000


def _tables(h, wq, bq, wka, wvv, bvbe):
    qt = (h @ wq + bq) * 0.25
    avt = jnp.concatenate([h @ wka, h @ wvv + bvbe], axis=1)
    return qt, avt


def _combined(ud, h, wb, wr, br):
    u = ud[:, :_NHID]
    d = ud[:, _NHID]
    b = h @ wb
    hr = h @ wr + br
    safe = d > 0.0
    dn = jnp.where(safe, d, 1.0)
    agg = jnp.where(safe[:, None], (u + b * d[:, None]) / dn[:, None], 0.0)
    return jnp.maximum(agg + hr, 0.0)


def _lin_in_node_body(x_ref, w0_ref, b0_ref, w1_ref, b1_ref,
                      wq_ref, bq_ref, wka_ref, wvv_ref, bvbe_ref,
                      h_ref, qt_ref, avt_ref):
    t = jnp.maximum(x_ref[...] @ w0_ref[...] + b0_ref[...], 0.0)
    h = jnp.maximum(t @ w1_ref[...] + b1_ref[...], 0.0)
    h_ref[...] = h
    qt, avt = _tables(h, wq_ref[...], bq_ref[...], wka_ref[...],
                      wvv_ref[...], bvbe_ref[...])
    qt_ref[...] = qt
    avt_ref[...] = avt


def _lin_in_node(x, W0, b0, W1, b1, Wq, bq, WkA, WvV, bvbe):
    grid = _N // _BN
    w16 = pl.BlockSpec((_NHID, _NHID), lambda i: (0, 0))
    b16 = pl.BlockSpec((1, _NHID), lambda i: (0, 0))
    return pl.pallas_call(
        _lin_in_node_body,
        grid=(grid,),
        in_specs=[
            pl.BlockSpec((_BN, 128), lambda i: (i, 0)),
            pl.BlockSpec((128, 128), lambda i: (0, 0)),
            pl.BlockSpec((1, 128), lambda i: (0, 0)),
            pl.BlockSpec((128, _NHID), lambda i: (0, 0)),
            b16, w16, b16, w16, w16, b16,
        ],
        out_specs=[
            pl.BlockSpec((_BN, _NHID), lambda i: (i, 0)),
            pl.BlockSpec((_BN, _NHID), lambda i: (i, 0)),
            pl.BlockSpec((_BN, 2 * _NHID), lambda i: (i, 0)),
        ],
        out_shape=[
            jax.ShapeDtypeStruct((_N, _NHID), jnp.float32),
            jax.ShapeDtypeStruct((_N, _NHID), jnp.float32),
            jax.ShapeDtypeStruct((_N, 2 * _NHID), jnp.float32),
        ],
    )(x, W0, b0.reshape(1, 128), W1, b1.reshape(1, _NHID),
      Wq, bq.reshape(1, _NHID), WkA, WvV, bvbe.reshape(1, _NHID))


def _combine_node_body(ud_ref, h_ref, wb_ref, wr_ref, br_ref,
                       wq_ref, bq_ref, wka_ref, wvv_ref, bvbe_ref,
                       h2_ref, qt_ref, avt_ref):
    ud = jnp.sum(ud_ref[...], axis=0)
    h2 = _combined(ud, h_ref[...], wb_ref[...], wr_ref[...], br_ref[...])
    h2_ref[...] = h2
    qt, avt = _tables(h2, wq_ref[...], bq_ref[...], wka_ref[...],
                      wvv_ref[...], bvbe_ref[...])
    qt_ref[...] = qt
    avt_ref[...] = avt


def _combine_node(UD, h, WB, Wr, br, Wq, bq, WkA, WvV, bvbe):
    grid = _N // _BN
    nu = UD.shape[0]
    w16 = pl.BlockSpec((_NHID, _NHID), lambda i: (0, 0))
    b16 = pl.BlockSpec((1, _NHID), lambda i: (0, 0))
    return pl.pallas_call(
        _combine_node_body,
        grid=(grid,),
        in_specs=[
            pl.BlockSpec((nu, _BN, 2 * _NHID), lambda i: (0, i, 0)),
            pl.BlockSpec((_BN, _NHID), lambda i: (i, 0)),
            w16, w16, b16, w16, b16, w16, w16, b16,
        ],
        out_specs=[
            pl.BlockSpec((_BN, _NHID), lambda i: (i, 0)),
            pl.BlockSpec((_BN, _NHID), lambda i: (i, 0)),
            pl.BlockSpec((_BN, 2 * _NHID), lambda i: (i, 0)),
        ],
        out_shape=[
            jax.ShapeDtypeStruct((_N, _NHID), jnp.float32),
            jax.ShapeDtypeStruct((_N, _NHID), jnp.float32),
            jax.ShapeDtypeStruct((_N, 2 * _NHID), jnp.float32),
        ],
    )(UD, h, WB, Wr, br.reshape(1, _NHID),
      Wq, bq.reshape(1, _NHID), WkA, WvV, bvbe.reshape(1, _NHID))


def _combine_proj_body(ud_ref, h_ref, wb_ref, wr_ref, br_ref,
                       wo_ref, bo_ref, o_ref):
    ud = jnp.sum(ud_ref[...], axis=0)
    h2 = _combined(ud, h_ref[...], wb_ref[...], wr_ref[...], br_ref[...])
    o_ref[...] = h2 @ wo_ref[...] + bo_ref[...]


def _combine_proj(UD, h, WB, Wr, br, Wout, bout):
    grid = _N // _BN
    nu = UD.shape[0]
    w16 = pl.BlockSpec((_NHID, _NHID), lambda i: (0, 0))
    b16 = pl.BlockSpec((1, _NHID), lambda i: (0, 0))
    return pl.pallas_call(
        _combine_proj_body,
        grid=(grid,),
        in_specs=[
            pl.BlockSpec((nu, _BN, 2 * _NHID), lambda i: (0, i, 0)),
            pl.BlockSpec((_BN, _NHID), lambda i: (i, 0)),
            w16, w16, b16,
            pl.BlockSpec((_NHID, 2), lambda i: (0, 0)),
            pl.BlockSpec((1, 2), lambda i: (0, 0)),
        ],
        out_specs=pl.BlockSpec((_BN, 2), lambda i: (i, 0)),
        out_shape=jax.ShapeDtypeStruct((_N, 2), jnp.float32),
    )(UD, h, WB, Wr, br.reshape(1, _NHID), Wout, bout.reshape(1, 2))


def _sc_mesh():
    return plsc.VectorSubcoreMesh(core_axis_name="c", subcore_axis_name="s")


@functools.cache
def _edge_sc_kernel():
    @functools.partial(
        pl.kernel,
        mesh=_sc_mesh(),
        out_type=jax.ShapeDtypeStruct((2, 16, _NP // 16, 2 * _NHID),
                                      jnp.float32),
        scratch_types=[
            pltpu.VMEM((_NCH, _KC), jnp.int32),
            pltpu.VMEM((_NCH, _KC), jnp.int32),
            pltpu.VMEM((2, _KC, _NHID), jnp.float32),
            pltpu.VMEM((2, _KC, 2 * _NHID), jnp.float32),
            pltpu.VMEM((2, _KC, 2 * _NHID), jnp.float32),
            pltpu.VMEM_SHARED((_NP, 2 * _NHID), jnp.float32),
            pltpu.SemaphoreType.DMA,
            pltpu.SemaphoreType.DMA,
            pltpu.SemaphoreType.DMA,
        ],
        compiler_params=pltpu.CompilerParams(use_tc_tiling_on_sc=False,
                                             needs_layout_passes=False),
    )
    def ek(qt_hbm, avt_hbm, srcr_hbm, dstr_hbm, zero_hbm, out_hbm,
           sidx, didx, qbuf, avbuf, mbuf, acc, semq, sema, semm):
        c = lax.axis_index("c")
        s = lax.axis_index("s")
        wid = s * 2 + c

        @pl.when(s == 0)
        def _():
            pltpu.sync_copy(zero_hbm, acc)

        plsc.subcore_barrier()
        pltpu.sync_copy(srcr_hbm.at[wid], sidx)
        pltpu.sync_copy(dstr_hbm.at[wid], didx)

        pltpu.async_copy(qt_hbm.at[didx.at[0]], qbuf.at[0], semq)
        pltpu.async_copy(avt_hbm.at[sidx.at[0]], avbuf.at[0], sema)

        def chunk(j, carry):
            sl = j & 1
            nxt = 1 - sl

            pltpu.make_async_copy(qt_hbm.at[didx.at[j]], qbuf.at[sl],
                                  semq).wait()
            pltpu.make_async_copy(avt_hbm.at[sidx.at[j]], avbuf.at[sl],
                                  sema).wait()

            @pl.when(j + 1 < _NCH)
            def _():
                pltpu.async_copy(qt_hbm.at[didx.at[j + 1]], qbuf.at[nxt],
                                 semq)
                pltpu.async_copy(avt_hbm.at[sidx.at[j + 1]], avbuf.at[nxt],
                                 sema)

            lane15 = jnp.full((_NHID, 1), _NHID - 1, jnp.int32)
            dnums = lax.GatherDimensionNumbers(
                offset_dims=(), collapsed_slice_dims=(0,),
                start_index_map=(0,))

            @plsc.parallel_loop(0, _KC, 1, unroll=10)
            def body(i):
                q = qbuf[sl, i, :]
                a = avbuf[sl, i, pl.ds(0, _NHID)]
                v = avbuf[sl, i, pl.ds(_NHID, _NHID)]
                acc_l = lax.cumsum(q * a)
                logit = lax.gather(
                    acc_l, lane15, dnums, (1,),
                    mode=lax.GatherScatterMode.PROMISE_IN_BOUNDS)
                p = jnp.exp(jnp.minimum(logit, 60.0))
                mbuf[sl, i, pl.ds(0, _NHID)] = p * v
                mbuf[sl, i, pl.ds(_NHID, _NHID)] = p

            @pl.when(j > 0)
            def _():
                pltpu.make_async_copy(zero_hbm.at[pl.ds(0, _KC)],
                                      mbuf.at[nxt], semm).wait()

            pltpu.async_copy(mbuf.at[sl], acc.at[didx.at[j]], semm,
                             add=True)
            return carry

        lax.fori_loop(0, _NCH, chunk, 0)
        pltpu.make_async_copy(zero_hbm.at[pl.ds(0, _KC)], mbuf.at[0],
                              semm).wait()
        plsc.subcore_barrier()
        rows = _NP // 16
        pltpu.sync_copy(acc.at[pl.ds(s * rows, rows)], out_hbm.at[c, s])

    return ek


def kernel(x, pos, norm, W0, b0, W1, b1, Wq, bq, Wk, bk, Wv, bv, We, be,
           Wr, br, Wout, bout, edge_index):
    srcr = edge_index[0].reshape(_NW, _NCH, _KC)
    dstr = edge_index[1].reshape(_NW, _NCH, _KC)
    zero = jnp.zeros((_NP, 2 * _NHID), jnp.float32)
    ek = _edge_sc_kernel()

    WkA = [Wk[l] + We[l][:_NHID] for l in range(_DEPTH)]
    WvV = [Wv[l] + We[l][:_NHID] for l in range(_DEPTH)]
    bvbe = [bv[l] + be[l] for l in range(_DEPTH)]
    WB = [We[l][_NHID:] for l in range(_DEPTH)]

    h, qt, avt = _lin_in_node(x, W0, b0, W1, b1,
                              Wq[0], bq[0], WkA[0], WvV[0], bvbe[0])
    for l in range(_DEPTH):
        UD4 = ek(qt, avt, srcr, dstr, zero)
        UD = UD4.reshape(2, _NP, 2 * _NHID)
        if l + 1 < _DEPTH:
            h, qt, avt = _combine_node(UD, h, WB[l], Wr[l], br[l],
                                       Wq[l + 1], bq[l + 1], WkA[l + 1],
                                       WvV[l + 1], bvbe[l + 1])
        else:
            out = _combine_proj(UD, h, WB[l], Wr[l], br[l], Wout, bout)
    return out


# speedup vs baseline: 1.2312x; 1.2312x over previous
import functools

import jax
import jax.numpy as jnp
from jax import lax
from jax.experimental import pallas as pl
from jax.experimental.pallas import tpu as pltpu
from jax.experimental.pallas import tpu_sc as plsc

_N = 10000
_E = 320000
_NHID = 16
_DEPTH = 3

_NW = 32
_KC = 100
_EPW = _E // _NW
_NCH = _EPW // _KC
_NP = 10240

_BN = 2000


def _tables(h, wq, bq, wka, wvv, bvbe):
    qt = (h @ wq + bq) * 0.25
    avt = jnp.concatenate([h @ wka, h @ wvv + bvbe], axis=1)
    return qt, avt


def _combined(ud, h, wb, wr, br):
    u = ud[:, :_NHID]
    d = ud[:, _NHID]
    b = h @ wb
    hr = h @ wr + br
    safe = d > 0.0
    dn = jnp.where(safe, d, 1.0)
    agg = jnp.where(safe[:, None], (u + b * d[:, None]) / dn[:, None], 0.0)
    return jnp.maximum(agg + hr, 0.0)


def _lin_in_node_body(x_ref, w0_ref, b0_ref, w1_ref, b1_ref,
                      wq_ref, bq_ref, wka_ref, wvv_ref, bvbe_ref,
                      h_ref, qt_ref, avt_ref):
    t = jnp.maximum(x_ref[...] @ w0_ref[...] + b0_ref[...], 0.0)
    h = jnp.maximum(t @ w1_ref[...] + b1_ref[...], 0.0)
    h_ref[...] = h
    qt, avt = _tables(h, wq_ref[...], bq_ref[...], wka_ref[...],
                      wvv_ref[...], bvbe_ref[...])
    qt_ref[...] = qt
    avt_ref[...] = avt


def _lin_in_node(x, W0, b0, W1, b1, Wq, bq, WkA, WvV, bvbe):
    grid = _N // _BN
    w16 = pl.BlockSpec((_NHID, _NHID), lambda i: (0, 0))
    b16 = pl.BlockSpec((1, _NHID), lambda i: (0, 0))
    return pl.pallas_call(
        _lin_in_node_body,
        grid=(grid,),
        in_specs=[
            pl.BlockSpec((_BN, 128), lambda i: (i, 0)),
            pl.BlockSpec((128, 128), lambda i: (0, 0)),
            pl.BlockSpec((1, 128), lambda i: (0, 0)),
            pl.BlockSpec((128, _NHID), lambda i: (0, 0)),
            b16, w16, b16, w16, w16, b16,
        ],
        out_specs=[
            pl.BlockSpec((_BN, _NHID), lambda i: (i, 0)),
            pl.BlockSpec((_BN, _NHID), lambda i: (i, 0)),
            pl.BlockSpec((_BN, 2 * _NHID), lambda i: (i, 0)),
        ],
        out_shape=[
            jax.ShapeDtypeStruct((_N, _NHID), jnp.float32),
            jax.ShapeDtypeStruct((_N, _NHID), jnp.float32),
            jax.ShapeDtypeStruct((_N, 2 * _NHID), jnp.float32),
        ],
    )(x, W0, b0.reshape(1, 128), W1, b1.reshape(1, _NHID),
      Wq, bq.reshape(1, _NHID), WkA, WvV, bvbe.reshape(1, _NHID))


def _combine_node_body(ud_ref, h_ref, wb_ref, wr_ref, br_ref,
                       wq_ref, bq_ref, wka_ref, wvv_ref, bvbe_ref,
                       h2_ref, qt_ref, avt_ref):
    ud = jnp.sum(ud_ref[...], axis=0)
    h2 = _combined(ud, h_ref[...], wb_ref[...], wr_ref[...], br_ref[...])
    h2_ref[...] = h2
    qt, avt = _tables(h2, wq_ref[...], bq_ref[...], wka_ref[...],
                      wvv_ref[...], bvbe_ref[...])
    qt_ref[...] = qt
    avt_ref[...] = avt


def _combine_node(UD, h, WB, Wr, br, Wq, bq, WkA, WvV, bvbe):
    grid = _N // _BN
    nu = UD.shape[0]
    w16 = pl.BlockSpec((_NHID, _NHID), lambda i: (0, 0))
    b16 = pl.BlockSpec((1, _NHID), lambda i: (0, 0))
    return pl.pallas_call(
        _combine_node_body,
        grid=(grid,),
        in_specs=[
            pl.BlockSpec((nu, _BN, 2 * _NHID), lambda i: (0, i, 0)),
            pl.BlockSpec((_BN, _NHID), lambda i: (i, 0)),
            w16, w16, b16, w16, b16, w16, w16, b16,
        ],
        out_specs=[
            pl.BlockSpec((_BN, _NHID), lambda i: (i, 0)),
            pl.BlockSpec((_BN, _NHID), lambda i: (i, 0)),
            pl.BlockSpec((_BN, 2 * _NHID), lambda i: (i, 0)),
        ],
        out_shape=[
            jax.ShapeDtypeStruct((_N, _NHID), jnp.float32),
            jax.ShapeDtypeStruct((_N, _NHID), jnp.float32),
            jax.ShapeDtypeStruct((_N, 2 * _NHID), jnp.float32),
        ],
    )(UD, h, WB, Wr, br.reshape(1, _NHID),
      Wq, bq.reshape(1, _NHID), WkA, WvV, bvbe.reshape(1, _NHID))


def _combine_proj_body(ud_ref, h_ref, wb_ref, wr_ref, br_ref,
                       wo_ref, bo_ref, o_ref):
    ud = jnp.sum(ud_ref[...], axis=0)
    h2 = _combined(ud, h_ref[...], wb_ref[...], wr_ref[...], br_ref[...])
    o_ref[...] = h2 @ wo_ref[...] + bo_ref[...]


def _combine_proj(UD, h, WB, Wr, br, Wout, bout):
    grid = _N // _BN
    nu = UD.shape[0]
    w16 = pl.BlockSpec((_NHID, _NHID), lambda i: (0, 0))
    b16 = pl.BlockSpec((1, _NHID), lambda i: (0, 0))
    return pl.pallas_call(
        _combine_proj_body,
        grid=(grid,),
        in_specs=[
            pl.BlockSpec((nu, _BN, 2 * _NHID), lambda i: (0, i, 0)),
            pl.BlockSpec((_BN, _NHID), lambda i: (i, 0)),
            w16, w16, b16,
            pl.BlockSpec((_NHID, 2), lambda i: (0, 0)),
            pl.BlockSpec((1, 2), lambda i: (0, 0)),
        ],
        out_specs=pl.BlockSpec((_BN, 2), lambda i: (i, 0)),
        out_shape=jax.ShapeDtypeStruct((_N, 2), jnp.float32),
    )(UD, h, WB, Wr, br.reshape(1, _NHID), Wout, bout.reshape(1, 2))


def _sc_mesh():
    return plsc.VectorSubcoreMesh(core_axis_name="c", subcore_axis_name="s")


@functools.cache
def _edge_sc_kernel():
    @functools.partial(
        pl.kernel,
        mesh=_sc_mesh(),
        out_type=jax.ShapeDtypeStruct((2, 16, _NP // 16, 2 * _NHID),
                                      jnp.float32),
        scratch_types=[
            pltpu.VMEM((_NCH, _KC), jnp.int32),
            pltpu.VMEM((_NCH, _KC), jnp.int32),
            pltpu.VMEM((2, _KC, _NHID), jnp.float32),
            pltpu.VMEM((2, _KC, 2 * _NHID), jnp.float32),
            pltpu.VMEM((2, _KC, 2 * _NHID), jnp.float32),
            pltpu.VMEM_SHARED((_NP, 2 * _NHID), jnp.float32),
            pltpu.SemaphoreType.DMA,
            pltpu.SemaphoreType.DMA,
            pltpu.SemaphoreType.DMA,
            pltpu.SemaphoreType.DMA,
            pltpu.SemaphoreType.DMA,
            pltpu.SemaphoreType.DMA,
        ],
        compiler_params=pltpu.CompilerParams(use_tc_tiling_on_sc=False,
                                             needs_layout_passes=False),
    )
    def ek(qt_hbm, avt_hbm, srcr_hbm, dstr_hbm, zero_hbm, out_hbm,
           sidx, didx, qbuf, avbuf, mbuf, acc,
           semq0, semq1, sema0, sema1, semm0, semm1):
        c = lax.axis_index("c")
        s = lax.axis_index("s")
        wid = s * 2 + c
        semq = (semq0, semq1)
        sema = (sema0, sema1)
        semm = (semm0, semm1)

        @pl.when(s == 0)
        def _():
            pltpu.sync_copy(zero_hbm, acc)

        plsc.subcore_barrier()
        pltpu.sync_copy(srcr_hbm.at[wid], sidx)
        pltpu.sync_copy(dstr_hbm.at[wid], didx)

        lane15 = jnp.full((_NHID, 1), _NHID - 1, jnp.int32)
        dnums = lax.GatherDimensionNumbers(
            offset_dims=(), collapsed_slice_dims=(0,),
            start_index_map=(0,))

        def fire(sl, j):
            pltpu.async_copy(qt_hbm.at[didx.at[j]], qbuf.at[sl], semq[sl])
            pltpu.async_copy(avt_hbm.at[sidx.at[j]], avbuf.at[sl], sema[sl])

        def compute(sl, j, j2):
            pltpu.make_async_copy(qt_hbm.at[didx.at[j]], qbuf.at[sl],
                                  semq[sl]).wait()
            pltpu.make_async_copy(avt_hbm.at[sidx.at[j]], avbuf.at[sl],
                                  sema[sl]).wait()

            @plsc.parallel_loop(0, _KC, 1, unroll=10)
            def body(i):
                q = qbuf[sl, i, :]
                a = avbuf[sl, i, pl.ds(0, _NHID)]
                v = avbuf[sl, i, pl.ds(_NHID, _NHID)]
                acc_l = lax.cumsum(q * a)
                logit = lax.gather(
                    acc_l, lane15, dnums, (1,),
                    mode=lax.GatherScatterMode.PROMISE_IN_BOUNDS)
                p = jnp.exp(jnp.minimum(logit, 60.0))
                mbuf[sl, i, pl.ds(0, _NHID)] = p * v
                mbuf[sl, i, pl.ds(_NHID, _NHID)] = p

            @pl.when(j2 > 0)
            def _():
                pltpu.make_async_copy(zero_hbm.at[pl.ds(0, _KC)],
                                      mbuf.at[sl], semm[sl]).wait()

            pltpu.async_copy(mbuf.at[sl], acc.at[didx.at[j]], semm[sl],
                             add=True)

        fire(0, 0)
        fire(1, 1)

        def chunk2(j2, carry):
            j0 = 2 * j2
            compute(0, j0, j2)

            @pl.when(j0 + 2 < _NCH)
            def _():
                fire(0, j0 + 2)

            compute(1, j0 + 1, j2)

            @pl.when(j0 + 3 < _NCH)
            def _():
                fire(1, j0 + 3)

            return carry

        lax.fori_loop(0, _NCH // 2, chunk2, 0)
        pltpu.make_async_copy(zero_hbm.at[pl.ds(0, _KC)], mbuf.at[0],
                              semm0).wait()
        pltpu.make_async_copy(zero_hbm.at[pl.ds(0, _KC)], mbuf.at[1],
                              semm1).wait()
        plsc.subcore_barrier()
        rows = _NP // 16
        pltpu.sync_copy(acc.at[pl.ds(s * rows, rows)], out_hbm.at[c, s])

    return ek


def kernel(x, pos, norm, W0, b0, W1, b1, Wq, bq, Wk, bk, Wv, bv, We, be,
           Wr, br, Wout, bout, edge_index):
    srcr = edge_index[0].reshape(_NW, _NCH, _KC)
    dstr = edge_index[1].reshape(_NW, _NCH, _KC)
    zero = jnp.zeros((_NP, 2 * _NHID), jnp.float32)
    ek = _edge_sc_kernel()

    WkA = [Wk[l] + We[l][:_NHID] for l in range(_DEPTH)]
    WvV = [Wv[l] + We[l][:_NHID] for l in range(_DEPTH)]
    bvbe = [bv[l] + be[l] for l in range(_DEPTH)]
    WB = [We[l][_NHID:] for l in range(_DEPTH)]

    h, qt, avt = _lin_in_node(x, W0, b0, W1, b1,
                              Wq[0], bq[0], WkA[0], WvV[0], bvbe[0])
    for l in range(_DEPTH):
        UD4 = ek(qt, avt, srcr, dstr, zero)
        UD = UD4.reshape(2, _NP, 2 * _NHID)
        if l + 1 < _DEPTH:
            h, qt, avt = _combine_node(UD, h, WB[l], Wr[l], br[l],
                                       Wq[l + 1], bq[l + 1], WkA[l + 1],
                                       WvV[l + 1], bvbe[l + 1])
        else:
            out = _combine_proj(UD, h, WB[l], Wr[l], br[l], Wout, bout)
    return out
